# Initial kernel scaffold; baseline (speedup 1.0000x reference)
import jax
import jax.numpy as jnp
from jax.experimental import pallas as pl


def kernel(batch_users, edge_rows, edge_cols, edge_vals, left_mat, right_mat):
    raise NotImplementedError("write your pallas kernel here")


# speedup vs baseline: 1.7165x; 1.7165x over previous
import functools

import jax
import jax.numpy as jnp
from jax.experimental import pallas as pl

NUM_USER = 30000
FACTOR = 256
IDL_BETA = 0.3


def _idl_c_kernel(left_ref, at_ref, c_ref):
    @pl.when(pl.program_id(0) == 0)
    def _():
        c_ref[...] = jnp.zeros_like(c_ref)

    c_ref[...] += jax.lax.dot_general(
        left_ref[...], at_ref[...], (((0,), (0,)), ((), ())),
        preferred_element_type=jnp.float32)


def _idl_out_kernel(right_ref, c_ref, out_ref):
    out_ref[...] = jax.lax.dot_general(
        right_ref[...], c_ref[...], (((0,), (0,)), ((), ())),
        preferred_element_type=jnp.float32)


def _idl_t(left_i, right_mat, at_b):
    ii, f = left_i.shape
    n = right_mat.shape[1]
    b = at_b.shape[1]
    ti = 2000
    c = pl.pallas_call(
        _idl_c_kernel,
        grid=(ii // ti,),
        in_specs=[pl.BlockSpec((ti, f), lambda i: (i, 0)),
                  pl.BlockSpec((ti, b), lambda i: (i, 0))],
        out_specs=pl.BlockSpec((f, b), lambda i: (0, 0)),
        out_shape=jax.ShapeDtypeStruct((f, b), jnp.float32),
    )(left_i, at_b)
    tn = 4096
    return pl.pallas_call(
        _idl_out_kernel,
        grid=(pl.cdiv(n, tn),),
        in_specs=[pl.BlockSpec((f, tn), lambda i: (0, i)),
                  pl.BlockSpec((f, b), lambda i: (0, 0))],
        out_specs=pl.BlockSpec((tn, b), lambda i: (i, 0)),
        out_shape=jax.ShapeDtypeStruct((n, b), jnp.float32),
    )(right_mat, c)


def kernel(batch_users, edge_rows, edge_cols, edge_vals, left_mat, right_mat):
    e2 = edge_rows.shape[0]
    e = e2 // 2
    n, f = left_mat.shape
    u_n = NUM_USER
    i_n = n - u_n
    b = batch_users.shape[0]

    eu = edge_rows[:e]
    ei = edge_cols[:e] - u_n
    ev = edge_vals[:e]

    pos = jnp.full((u_n,), b, jnp.int32).at[batch_users].set(
        jnp.arange(b, dtype=jnp.int32))
    p = pos[eu]
    at_b = jnp.zeros((i_n, b + 1), jnp.float32).at[ei, p].set(ev)[:, :b]

    idl_t = _idl_t(left_mat[u_n:], right_mat, at_b)

    def spmm_ui(x):
        return jax.ops.segment_sum(x[ei] * ev[:, None], eu, num_segments=u_n)

    def spmm_iu(x):
        return jax.ops.segment_sum(x[eu] * ev[:, None], ei, num_segments=i_n)

    s1 = spmm_ui(at_b)
    blur_it = spmm_iu(s1)
    yt_u = IDL_BETA * idl_t[:u_n]
    yt_i = IDL_BETA * idl_t[u_n:] + blur_it
    out_ut = yt_u - spmm_ui(spmm_iu(yt_u))
    out_it = yt_i - spmm_iu(spmm_ui(yt_i))
    return jnp.concatenate([out_ut, out_it], axis=0).T
